# baseline (speedup 1.0000x reference)
import functools

import jax
import jax.numpy as jnp
from jax import lax
from jax.experimental import pallas as pl
from jax.experimental.pallas import tpu as pltpu
from jax.experimental.pallas import tpu_sc as plsc

NSTEP = 5
H = 256
DIN = 128
N = 10000
E = 640000
G = 256
EPS = 1e-5

NC = 2
NS = 16
L = 16
NSTR = NC * NS
SPT = 320
OUT_ROWS = NSTR * SPT + NSTR
EPT = E // NSTR
CAP2 = 20160
CHUNK = 80
ZR = 128

_NLP = pltpu.CompilerParams(needs_layout_passes=False)


def _partition_body(esrc_hbm, edst_hbm, src_out, dst_out, nch_out,
                    in_src, in_dst, cm_src, cm_dst, stage16):
    c = lax.axis_index("c")
    s = lax.axis_index("s")
    u = c * NS + s
    lanes = lax.iota(jnp.int32, L)

    pltpu.sync_copy(esrc_hbm.at[pl.ds(u * EPT, EPT)], in_src)
    pltpu.sync_copy(edst_hbm.at[pl.ds(u * EPT, EPT)], in_dst)

    def stripe(t, _):
        lo = t * SPT

        def vec(v, off):
            sv = in_src[pl.ds(v * L, L)]
            dv = in_dst[pl.ds(v * L, L)]
            mask = (dv >= lo) & (dv < lo + SPT)
            pos = plsc.cumsum(mask.astype(jnp.int32))
            idx = off + pos - 1
            plsc.store_scatter(cm_src, [idx], sv, mask=mask)
            plsc.store_scatter(cm_dst, [idx], dv, mask=mask)
            return off + jnp.max(pos)

        off = lax.fori_loop(0, EPT // L, vec, jnp.int32(0))

        dummy = jnp.full((L,), t * SPT + SPT, jnp.int32)
        zsrc = jnp.zeros((L,), jnp.int32)
        for p in range(CHUNK // L):
            pidx = off + p * L + lanes
            plsc.store_scatter(cm_src, [pidx], zsrc)
            plsc.store_scatter(cm_dst, [pidx], dummy)
        off_pad = ((off + CHUNK - 1) // CHUNK) * CHUNK

        b = u * NSTR + t
        pltpu.sync_copy(cm_src, src_out.at[pl.ds(b * CAP2, CAP2)])
        pltpu.sync_copy(cm_dst, dst_out.at[pl.ds(b * CAP2, CAP2)])
        stage16[...] = jnp.full((L,), off_pad // CHUNK, jnp.int32)
        pltpu.sync_copy(stage16, nch_out.at[pl.ds(b * L, L)])
        return 0

    lax.fori_loop(0, NSTR, stripe, 0)


@functools.cache
def _sc_mesh():
    return plsc.VectorSubcoreMesh(
        core_axis_name="c", subcore_axis_name="s",
        num_cores=NC, num_subcores=NS,
    )


@functools.cache
def _partition_kernel():
    return pl.kernel(
        _partition_body,
        out_type=(
            jax.ShapeDtypeStruct((NSTR * NSTR * CAP2,), jnp.int32),
            jax.ShapeDtypeStruct((NSTR * NSTR * CAP2,), jnp.int32),
            jax.ShapeDtypeStruct((NSTR * NSTR * L,), jnp.int32),
        ),
        mesh=_sc_mesh(),
        compiler_params=_NLP,
        scratch_types=[
            pltpu.VMEM((EPT,), jnp.int32),
            pltpu.VMEM((EPT,), jnp.int32),
            pltpu.VMEM((CAP2,), jnp.int32),
            pltpu.VMEM((CAP2,), jnp.int32),
            pltpu.VMEM((L,), jnp.int32),
        ],
    )


SPT_P = SPT + 1


def _aggregate_body(h_hbm, src_l, dst_l, nch_tbl, z_hbm, out_hbm,
                    acc, idx_s0, idx_d0, rows0, idx_s1, idx_d1, rows1,
                    nch_v, sem0, sem1):
    c = lax.axis_index("c")
    s = lax.axis_index("s")
    t = c * NS + s
    lanes = lax.iota(jnp.int32, L)

    pltpu.sync_copy(z_hbm, acc)

    slots = ((idx_s0, idx_d0, rows0, sem0), (idx_s1, idx_d1, rows1, sem1))

    def load_start(b, j, slot):
        i_s, i_d, rw, sm = slots[slot]
        base = b * CAP2 + j * CHUNK
        pltpu.sync_copy(src_l.at[pl.ds(base, CHUNK)], i_s)
        pltpu.sync_copy(dst_l.at[pl.ds(base, CHUNK)], i_d)
        pltpu.async_copy(h_hbm.at[i_s], rw, sm)

    def accum(slot):
        i_s, i_d, rw, sm = slots[slot]
        pltpu.make_async_copy(h_hbm.at[i_s], rw, sm).wait()

        def grp(j2, _):
            dv = i_d[pl.ds(j2 * L, L)] - t * SPT
            for p in range(L):
                rbase = dv[p] * H
                for k in range(H // L):
                    v = rw[j2 * L + p, pl.ds(k * L, L)]
                    plsc.addupdate_scatter(acc, [rbase + k * L + lanes], v)
            return 0

        lax.fori_loop(0, CHUNK // L, grp, 0)

    def bucket(u, _):
        b = u * NSTR + t
        pltpu.sync_copy(nch_tbl.at[pl.ds(b * L, L)], nch_v)
        n = jnp.max(nch_v[...])

        @pl.when(n > 0)
        def _():
            load_start(b, 0, 0)

        def pair(jj, _):
            j0 = 2 * jj

            @pl.when(j0 + 1 < n)
            def _():
                load_start(b, j0 + 1, 1)

            accum(0)

            @pl.when(j0 + 2 < n)
            def _():
                load_start(b, j0 + 2, 0)

            @pl.when(j0 + 1 < n)
            def _():
                accum(1)

            return 0

        lax.fori_loop(0, (n + 1) // 2, pair, 0)
        return 0

    lax.fori_loop(0, NSTR, bucket, 0)

    pltpu.sync_copy(acc.at[pl.ds(0, SPT * H)],
                    out_hbm.at[pl.ds(t * SPT * H, SPT * H)])


@functools.cache
def _aggregate_kernel():
    return pl.kernel(
        _aggregate_body,
        out_type=jax.ShapeDtypeStruct((NSTR * SPT * H,), jnp.float32),
        mesh=_sc_mesh(),
        compiler_params=_NLP,
        scratch_types=[
            pltpu.VMEM((SPT_P * H,), jnp.float32),
            pltpu.VMEM((CHUNK,), jnp.int32),
            pltpu.VMEM((CHUNK,), jnp.int32),
            pltpu.VMEM((CHUNK, H), jnp.float32),
            pltpu.VMEM((CHUNK,), jnp.int32),
            pltpu.VMEM((CHUNK,), jnp.int32),
            pltpu.VMEM((CHUNK, H), jnp.float32),
            pltpu.VMEM((L,), jnp.int32),
            pltpu.SemaphoreType.DMA,
            pltpu.SemaphoreType.DMA,
        ],
    )


def _bn_relu(h, g, b):
    mu = jnp.mean(h, axis=0, keepdims=True)
    var = jnp.mean((h - mu) ** 2, axis=0, keepdims=True)
    return jnp.maximum((h - mu) / jnp.sqrt(var + EPS) * g + b, 0.0)


def _input_body(x_ref, w_ref, b_ref, o_ref):
    o_ref[...] = jnp.dot(x_ref[...], w_ref[...],
                         preferred_element_type=jnp.float32) + b_ref[...]


_input_layer = pl.pallas_call(
    _input_body,
    out_shape=jax.ShapeDtypeStruct((N, H), jnp.float32),
)


def _msg_body(h_ref, w_ref, o_ref):
    o_ref[...] = jnp.dot(h_ref[...], w_ref[...],
                         preferred_element_type=jnp.float32)


_msg = pl.pallas_call(
    _msg_body,
    out_shape=jax.ShapeDtypeStruct((N, H), jnp.float32),
)


def _gru_body(aggr_ref, h_ref, wih_ref, whh_ref, bih_ref, bhh_ref, o_ref):
    h = h_ref[...]
    aggr = aggr_ref[...]
    gi = lax.dot_general(aggr, wih_ref[...], (((1,), (1,)), ((), ())),
                         preferred_element_type=jnp.float32) + bih_ref[...]
    gh = lax.dot_general(h, whh_ref[...], (((1,), (1,)), ((), ())),
                         preferred_element_type=jnp.float32) + bhh_ref[...]
    r = jax.nn.sigmoid(gi[:, :H] + gh[:, :H])
    z = jax.nn.sigmoid(gi[:, H:2 * H] + gh[:, H:2 * H])
    nn = jnp.tanh(gi[:, 2 * H:] + r * gh[:, 2 * H:])
    o_ref[...] = (1.0 - z) * nn + z * h


_BR = 2000

_gru_step = pl.pallas_call(
    _gru_body,
    grid=(N // _BR,),
    in_specs=[
        pl.BlockSpec((_BR, H), lambda i: (i, 0)),
        pl.BlockSpec((_BR, H), lambda i: (i, 0)),
        pl.BlockSpec((3 * H, H), lambda i: (0, 0)),
        pl.BlockSpec((3 * H, H), lambda i: (0, 0)),
        pl.BlockSpec((1, 3 * H), lambda i: (0, 0)),
        pl.BlockSpec((1, 3 * H), lambda i: (0, 0)),
    ],
    out_specs=pl.BlockSpec((_BR, H), lambda i: (i, 0)),
    out_shape=jax.ShapeDtypeStruct((N, H), jnp.float32),
)


def _head_body(h_ref, brow_ref, bcol_ref, g2_ref, b2_ref,
               f1w_ref, f1b_ref, g3_ref, b3_ref,
               f2w_ref, f2b_ref, g4_ref, b4_ref,
               f3w_ref, f3b_ref, o_ref, xmax_ref):
    h = _bn_relu(h_ref[...], g2_ref[...], b2_ref[...])
    brow = brow_ref[...]
    bcol = bcol_ref[...]

    gids = lax.broadcasted_iota(jnp.int32, (G, N), 0)
    onehot = (gids == brow).astype(jnp.float32)
    sums = jnp.dot(onehot, h, preferred_element_type=jnp.float32,
                   precision=lax.Precision.HIGHEST)
    counts = jnp.sum(onehot, axis=1, keepdims=True)
    x_mean = sums / jnp.maximum(counts, 1.0)

    def gmax(g, _):
        mask = bcol == g
        mx = jnp.max(jnp.where(mask, h, -jnp.inf), axis=0, keepdims=True)
        xmax_ref[pl.ds(g, 1), :] = mx
        return 0
    lax.fori_loop(0, G, gmax, 0)
    x_max = jnp.where(counts > 0.0, xmax_ref[...], 0.0)

    zc = jnp.concatenate([x_mean, x_max], axis=1)
    z1 = _bn_relu(jnp.dot(zc, f1w_ref[...], preferred_element_type=jnp.float32)
                  + f1b_ref[...], g3_ref[...], b3_ref[...])
    z2 = _bn_relu(jnp.dot(z1, f2w_ref[...], preferred_element_type=jnp.float32)
                  + f2b_ref[...], g4_ref[...], b4_ref[...])
    o_ref[...] = jnp.dot(z2, f3w_ref[...], preferred_element_type=jnp.float32) \
        + f3b_ref[...]


_head = pl.pallas_call(
    _head_body,
    out_shape=jax.ShapeDtypeStruct((G, 128), jnp.float32),
    scratch_shapes=[pltpu.VMEM((G, H), jnp.float32)],
)


def kernel(x, W_in, b_in, bn1_g, bn1_b, ggc_W, gru_Wih, gru_Whh, gru_bih,
           gru_bhh, bn2_g, bn2_b, fc1_W, fc1_b, bn3_g, bn3_b, fc2_W, fc2_b,
           bn4_g, bn4_b, fc3_W, fc3_b, edge_index, batch):
    row = lambda v: v.reshape(1, -1)

    hpre = x @ W_in + b_in
    mu1 = jnp.mean(hpre, axis=0)
    var1 = jnp.mean((hpre - mu1) ** 2, axis=0)
    h = jax.nn.relu((hpre - mu1) / jnp.sqrt(var1 + EPS) * bn1_g + bn1_b)

    src_l, dst_l, nch = _partition_kernel()(edge_index[0], edge_index[1])
    zeros = jnp.zeros((SPT_P * H,), jnp.float32)

    for i in range(NSTEP):
        m = _msg(h, ggc_W[i])
        aggr = _aggregate_kernel()(m, src_l, dst_l, nch, zeros)
        aggr = aggr.reshape(NSTR * SPT, H)[:N]
        h = _gru_step(aggr, h, gru_Wih, gru_Whh,
                      row(gru_bih), row(gru_bhh))

    f3w = jnp.zeros((H // 2, 128), jnp.float32).at[:, :2].set(fc3_W)
    f3b = jnp.zeros((1, 128), jnp.float32).at[:, :2].set(fc3_b.reshape(1, -1))

    out = _head(h, batch.reshape(1, N), batch.reshape(N, 1),
                row(bn2_g), row(bn2_b), fc1_W, row(fc1_b), row(bn3_g),
                row(bn3_b), fc2_W, row(fc2_b), row(bn4_g), row(bn4_b),
                f3w, f3b)
    return out[:, :2]


# speedup vs baseline: 1.1221x; 1.1221x over previous
import functools

import jax
import jax.numpy as jnp
from jax import lax
from jax.experimental import pallas as pl
from jax.experimental.pallas import tpu as pltpu
from jax.experimental.pallas import tpu_sc as plsc

NSTEP = 5
H = 256
DIN = 128
N = 10000
E = 640000
G = 256
EPS = 1e-5

NC = 2
NS = 16
L = 16
NSTR = NC * NS
SPT = 320
OUT_ROWS = NSTR * SPT + NSTR
EPT = E // NSTR
CAP2 = 20160
CHUNK = 48
ZR = 128

_NLP = pltpu.CompilerParams(needs_layout_passes=False)


def _partition_body(esrc_hbm, edst_hbm, src_out, dst_out, nch_out,
                    in_src, in_dst, cm_src, cm_dst, stage16):
    c = lax.axis_index("c")
    s = lax.axis_index("s")
    u = c * NS + s
    lanes = lax.iota(jnp.int32, L)

    pltpu.sync_copy(esrc_hbm.at[pl.ds(u * EPT, EPT)], in_src)
    pltpu.sync_copy(edst_hbm.at[pl.ds(u * EPT, EPT)], in_dst)

    def stripe(t, _):
        lo = t * SPT

        def vec(v, off):
            sv = in_src[pl.ds(v * L, L)]
            dv = in_dst[pl.ds(v * L, L)]
            mask = (dv >= lo) & (dv < lo + SPT)
            pos = plsc.cumsum(mask.astype(jnp.int32))
            idx = off + pos - 1
            plsc.store_scatter(cm_src, [idx], sv, mask=mask)
            plsc.store_scatter(cm_dst, [idx], dv, mask=mask)
            return off + jnp.max(pos)

        off = lax.fori_loop(0, EPT // L, vec, jnp.int32(0))

        dummy = jnp.full((L,), t * SPT + SPT, jnp.int32)
        zsrc = jnp.zeros((L,), jnp.int32)
        for p in range(CHUNK // L):
            pidx = off + p * L + lanes
            plsc.store_scatter(cm_src, [pidx], zsrc)
            plsc.store_scatter(cm_dst, [pidx], dummy)
        off_pad = ((off + CHUNK - 1) // CHUNK) * CHUNK

        b = u * NSTR + t
        pltpu.sync_copy(cm_src, src_out.at[pl.ds(b * CAP2, CAP2)])
        pltpu.sync_copy(cm_dst, dst_out.at[pl.ds(b * CAP2, CAP2)])
        stage16[...] = jnp.full((L,), off_pad // CHUNK, jnp.int32)
        pltpu.sync_copy(stage16, nch_out.at[pl.ds(b * L, L)])
        return 0

    lax.fori_loop(0, NSTR, stripe, 0)


@functools.cache
def _sc_mesh():
    return plsc.VectorSubcoreMesh(
        core_axis_name="c", subcore_axis_name="s",
        num_cores=NC, num_subcores=NS,
    )


@functools.cache
def _partition_kernel():
    return pl.kernel(
        _partition_body,
        out_type=(
            jax.ShapeDtypeStruct((NSTR * NSTR * CAP2,), jnp.int32),
            jax.ShapeDtypeStruct((NSTR * NSTR * CAP2,), jnp.int32),
            jax.ShapeDtypeStruct((NSTR * NSTR * L,), jnp.int32),
        ),
        mesh=_sc_mesh(),
        compiler_params=_NLP,
        scratch_types=[
            pltpu.VMEM((EPT,), jnp.int32),
            pltpu.VMEM((EPT,), jnp.int32),
            pltpu.VMEM((CAP2,), jnp.int32),
            pltpu.VMEM((CAP2,), jnp.int32),
            pltpu.VMEM((L,), jnp.int32),
        ],
    )


SPT_P = SPT + 1


def _aggregate_body(h_hbm, src_l, dst_l, nch_tbl, z_hbm, out_hbm,
                    acc, idx_s0, idx_d0, rows0, idx_s1, idx_d1, rows1,
                    nch_v, sem0, sem1):
    c = lax.axis_index("c")
    s = lax.axis_index("s")
    t = c * NS + s
    lanes = lax.iota(jnp.int32, L)

    pltpu.sync_copy(z_hbm, acc)

    slots = ((idx_s0, idx_d0, rows0, sem0), (idx_s1, idx_d1, rows1, sem1))

    def load_start(b, j, slot):
        i_s, i_d, rw, sm = slots[slot]
        base = b * CAP2 + j * CHUNK
        pltpu.sync_copy(src_l.at[pl.ds(base, CHUNK)], i_s)
        pltpu.sync_copy(dst_l.at[pl.ds(base, CHUNK)], i_d)
        pltpu.async_copy(h_hbm.at[i_s], rw, sm)

    def accum(slot):
        i_s, i_d, rw, sm = slots[slot]
        pltpu.make_async_copy(h_hbm.at[i_s], rw, sm).wait()

        def grp(j2, _):
            dv = i_d[pl.ds(j2 * L, L)] - t * SPT
            for p in range(L):
                rbase = dv[p] * H
                for k in range(H // L):
                    v = rw[j2 * L + p, pl.ds(k * L, L)]
                    plsc.addupdate_scatter(acc, [rbase + k * L + lanes], v)
            return 0

        lax.fori_loop(0, CHUNK // L, grp, 0)

    def bucket(u, _):
        b = u * NSTR + t
        pltpu.sync_copy(nch_tbl.at[pl.ds(b * L, L)], nch_v)
        n = jnp.max(nch_v[...])

        @pl.when(n > 0)
        def _():
            load_start(b, 0, 0)

        def pair(jj, _):
            j0 = 2 * jj

            @pl.when(j0 + 1 < n)
            def _():
                load_start(b, j0 + 1, 1)

            accum(0)

            @pl.when(j0 + 2 < n)
            def _():
                load_start(b, j0 + 2, 0)

            @pl.when(j0 + 1 < n)
            def _():
                accum(1)

            return 0

        lax.fori_loop(0, (n + 1) // 2, pair, 0)
        return 0

    lax.fori_loop(0, NSTR, bucket, 0)

    pltpu.sync_copy(acc.at[pl.ds(0, SPT * H)],
                    out_hbm.at[pl.ds(t * SPT * H, SPT * H)])


@functools.cache
def _aggregate_kernel():
    return pl.kernel(
        _aggregate_body,
        out_type=jax.ShapeDtypeStruct((NSTR * SPT * H,), jnp.float32),
        mesh=_sc_mesh(),
        compiler_params=_NLP,
        scratch_types=[
            pltpu.VMEM((SPT_P * H,), jnp.float32),
            pltpu.VMEM((CHUNK,), jnp.int32),
            pltpu.VMEM((CHUNK,), jnp.int32),
            pltpu.VMEM((CHUNK, H), jnp.float32),
            pltpu.VMEM((CHUNK,), jnp.int32),
            pltpu.VMEM((CHUNK,), jnp.int32),
            pltpu.VMEM((CHUNK, H), jnp.float32),
            pltpu.VMEM((L,), jnp.int32),
            pltpu.SemaphoreType.DMA,
            pltpu.SemaphoreType.DMA,
        ],
    )


def _bn_relu(h, g, b):
    mu = jnp.mean(h, axis=0, keepdims=True)
    var = jnp.mean((h - mu) ** 2, axis=0, keepdims=True)
    return jnp.maximum((h - mu) / jnp.sqrt(var + EPS) * g + b, 0.0)


def _input_body(x_ref, w_ref, b_ref, o_ref):
    o_ref[...] = jnp.dot(x_ref[...], w_ref[...],
                         preferred_element_type=jnp.float32) + b_ref[...]


_input_layer = pl.pallas_call(
    _input_body,
    out_shape=jax.ShapeDtypeStruct((N, H), jnp.float32),
)


def _msg_body(h_ref, w_ref, o_ref):
    o_ref[...] = jnp.dot(h_ref[...], w_ref[...],
                         preferred_element_type=jnp.float32)


_msg = pl.pallas_call(
    _msg_body,
    out_shape=jax.ShapeDtypeStruct((N, H), jnp.float32),
)


def _gru_body(aggr_ref, h_ref, wih_ref, whh_ref, bih_ref, bhh_ref, o_ref):
    h = h_ref[...]
    aggr = aggr_ref[...]
    gi = lax.dot_general(aggr, wih_ref[...], (((1,), (1,)), ((), ())),
                         preferred_element_type=jnp.float32) + bih_ref[...]
    gh = lax.dot_general(h, whh_ref[...], (((1,), (1,)), ((), ())),
                         preferred_element_type=jnp.float32) + bhh_ref[...]
    r = jax.nn.sigmoid(gi[:, :H] + gh[:, :H])
    z = jax.nn.sigmoid(gi[:, H:2 * H] + gh[:, H:2 * H])
    nn = jnp.tanh(gi[:, 2 * H:] + r * gh[:, 2 * H:])
    o_ref[...] = (1.0 - z) * nn + z * h


_BR = 2000

_gru_step = pl.pallas_call(
    _gru_body,
    grid=(N // _BR,),
    in_specs=[
        pl.BlockSpec((_BR, H), lambda i: (i, 0)),
        pl.BlockSpec((_BR, H), lambda i: (i, 0)),
        pl.BlockSpec((3 * H, H), lambda i: (0, 0)),
        pl.BlockSpec((3 * H, H), lambda i: (0, 0)),
        pl.BlockSpec((1, 3 * H), lambda i: (0, 0)),
        pl.BlockSpec((1, 3 * H), lambda i: (0, 0)),
    ],
    out_specs=pl.BlockSpec((_BR, H), lambda i: (i, 0)),
    out_shape=jax.ShapeDtypeStruct((N, H), jnp.float32),
)


def _head_body(h_ref, brow_ref, bcol_ref, g2_ref, b2_ref,
               f1w_ref, f1b_ref, g3_ref, b3_ref,
               f2w_ref, f2b_ref, g4_ref, b4_ref,
               f3w_ref, f3b_ref, o_ref, xmax_ref):
    h = _bn_relu(h_ref[...], g2_ref[...], b2_ref[...])
    brow = brow_ref[...]
    bcol = bcol_ref[...]

    gids = lax.broadcasted_iota(jnp.int32, (G, N), 0)
    onehot = (gids == brow).astype(jnp.float32)
    sums = jnp.dot(onehot, h, preferred_element_type=jnp.float32,
                   precision=lax.Precision.HIGHEST)
    counts = jnp.sum(onehot, axis=1, keepdims=True)
    x_mean = sums / jnp.maximum(counts, 1.0)

    def gmax(g, _):
        mask = bcol == g
        mx = jnp.max(jnp.where(mask, h, -jnp.inf), axis=0, keepdims=True)
        xmax_ref[pl.ds(g, 1), :] = mx
        return 0
    lax.fori_loop(0, G, gmax, 0)
    x_max = jnp.where(counts > 0.0, xmax_ref[...], 0.0)

    zc = jnp.concatenate([x_mean, x_max], axis=1)
    z1 = _bn_relu(jnp.dot(zc, f1w_ref[...], preferred_element_type=jnp.float32)
                  + f1b_ref[...], g3_ref[...], b3_ref[...])
    z2 = _bn_relu(jnp.dot(z1, f2w_ref[...], preferred_element_type=jnp.float32)
                  + f2b_ref[...], g4_ref[...], b4_ref[...])
    o_ref[...] = jnp.dot(z2, f3w_ref[...], preferred_element_type=jnp.float32) \
        + f3b_ref[...]


_head = pl.pallas_call(
    _head_body,
    out_shape=jax.ShapeDtypeStruct((G, 128), jnp.float32),
    scratch_shapes=[pltpu.VMEM((G, H), jnp.float32)],
)


def kernel(x, W_in, b_in, bn1_g, bn1_b, ggc_W, gru_Wih, gru_Whh, gru_bih,
           gru_bhh, bn2_g, bn2_b, fc1_W, fc1_b, bn3_g, bn3_b, fc2_W, fc2_b,
           bn4_g, bn4_b, fc3_W, fc3_b, edge_index, batch):
    row = lambda v: v.reshape(1, -1)

    hpre = x @ W_in + b_in
    mu1 = jnp.mean(hpre, axis=0)
    var1 = jnp.mean((hpre - mu1) ** 2, axis=0)
    h = jax.nn.relu((hpre - mu1) / jnp.sqrt(var1 + EPS) * bn1_g + bn1_b)

    src_l, dst_l, nch = _partition_kernel()(edge_index[0], edge_index[1])
    zeros = jnp.zeros((SPT_P * H,), jnp.float32)

    for i in range(NSTEP):
        m = _msg(h, ggc_W[i])
        aggr = _aggregate_kernel()(m, src_l, dst_l, nch, zeros)
        aggr = aggr.reshape(NSTR * SPT, H)[:N]
        h = _gru_step(aggr, h, gru_Wih, gru_Whh,
                      row(gru_bih), row(gru_bhh))

    f3w = jnp.zeros((H // 2, 128), jnp.float32).at[:, :2].set(fc3_W)
    f3b = jnp.zeros((1, 128), jnp.float32).at[:, :2].set(fc3_b.reshape(1, -1))

    out = _head(h, batch.reshape(1, N), batch.reshape(N, 1),
                row(bn2_g), row(bn2_b), fc1_W, row(fc1_b), row(bn3_g),
                row(bn3_b), fc2_W, row(fc2_b), row(bn4_g), row(bn4_b),
                f3w, f3b)
    return out[:, :2]
